# Initial kernel scaffold; baseline (speedup 1.0000x reference)
import jax
import jax.numpy as jnp
from jax.experimental import pallas as pl


def kernel(x, x2, w_head, b_head, w_proj, b_proj):
    raise NotImplementedError("write your pallas kernel here")


# speedup vs baseline: 4.3850x; 4.3850x over previous
import functools

import jax
import jax.numpy as jnp
from jax import lax
from jax.experimental import pallas as pl
from jax.experimental.pallas import tpu as pltpu


def _pick_tile(b, target=256):
    best = 8
    for t in range(8, min(target, b) + 1, 8):
        if b % t == 0:
            best = t
    return best


def _body(x_ref, x2_ref, wh_ref, bh_ref, wp_ref, bp_ref,
          o1a_ref, o2a_ref, o1b_ref, o2b_ref, *, feat_dim, tb):
    xb = jnp.concatenate([x_ref[...], x2_ref[...]], axis=0).astype(jnp.bfloat16)
    f = jnp.dot(xb, wh_ref[...], preferred_element_type=jnp.float32)
    f = jnp.maximum(f + bh_ref[...], 0.0).astype(jnp.bfloat16)
    y = jnp.dot(f, wp_ref[...], preferred_element_type=jnp.float32) + bp_ref[...]
    y1 = y[:, :feat_dim]
    y2 = y[:, feat_dim:]
    n1 = y1 * lax.rsqrt(jnp.sum(y1 * y1, axis=-1, keepdims=True))
    n2 = y2 * lax.rsqrt(jnp.sum(y2 * y2, axis=-1, keepdims=True))
    o1a_ref[...] = n1[:tb].astype(o1a_ref.dtype)
    o2a_ref[...] = n2[:tb].astype(o2a_ref.dtype)
    o1b_ref[...] = n1[tb:].astype(o1b_ref.dtype)
    o2b_ref[...] = n2[tb:].astype(o2b_ref.dtype)


@jax.jit
def _run(x, x2, w_head, b_head, w_proj, b_proj):
    B, D = x.shape
    F2 = w_proj.shape[1]
    feat_dim = F2 // 2
    tb = _pick_tile(B)
    wh = w_head.astype(jnp.bfloat16)
    wp = w_proj.astype(jnp.bfloat16)
    out_block = pl.BlockSpec((tb, feat_dim), lambda i: (i, 0))
    return pl.pallas_call(
        functools.partial(_body, feat_dim=feat_dim, tb=tb),
        out_shape=tuple(jax.ShapeDtypeStruct((B, feat_dim), x.dtype)
                        for _ in range(4)),
        grid=(B // tb,),
        in_specs=[
            pl.BlockSpec((tb, D), lambda i: (i, 0)),
            pl.BlockSpec((tb, D), lambda i: (i, 0)),
            pl.BlockSpec((D, D), lambda i: (0, 0)),
            pl.BlockSpec((1, D), lambda i: (0, 0)),
            pl.BlockSpec((D, F2), lambda i: (0, 0)),
            pl.BlockSpec((1, F2), lambda i: (0, 0)),
        ],
        out_specs=(out_block, out_block, out_block, out_block),
        compiler_params=pltpu.CompilerParams(
            dimension_semantics=("parallel",),
            vmem_limit_bytes=64 * 1024 * 1024,
        ),
    )(x, x2, wh, b_head, wp, b_proj)


def kernel(x, x2, w_head, b_head, w_proj, b_proj):
    return _run(x, x2, w_head, b_head, w_proj, b_proj)


# speedup vs baseline: 4.5081x; 1.0281x over previous
import functools

import jax
import jax.numpy as jnp
from jax import lax
from jax.experimental import pallas as pl
from jax.experimental.pallas import tpu as pltpu


def _pick_tile(b, target=512):
    best = 8
    for t in range(8, min(target, b) + 1, 8):
        if b % t == 0:
            best = t
    return best


def _body(x_ref, x2_ref, wh_ref, bh_ref, wp_ref, bp_ref,
          o1a_ref, o2a_ref, o1b_ref, o2b_ref, *, feat_dim, tb):
    xb = jnp.concatenate([x_ref[...], x2_ref[...]], axis=0).astype(jnp.bfloat16)
    f = jnp.dot(xb, wh_ref[...], preferred_element_type=jnp.float32)
    f = jnp.maximum(f + bh_ref[...], 0.0).astype(jnp.bfloat16)
    y = jnp.dot(f, wp_ref[...], preferred_element_type=jnp.float32) + bp_ref[...]
    y1 = y[:, :feat_dim]
    y2 = y[:, feat_dim:]
    n1 = y1 * lax.rsqrt(jnp.sum(y1 * y1, axis=-1, keepdims=True))
    n2 = y2 * lax.rsqrt(jnp.sum(y2 * y2, axis=-1, keepdims=True))
    o1a_ref[...] = n1[:tb].astype(o1a_ref.dtype)
    o2a_ref[...] = n2[:tb].astype(o2a_ref.dtype)
    o1b_ref[...] = n1[tb:].astype(o1b_ref.dtype)
    o2b_ref[...] = n2[tb:].astype(o2b_ref.dtype)


@jax.jit
def _run(x, x2, w_head, b_head, w_proj, b_proj):
    B, D = x.shape
    F2 = w_proj.shape[1]
    feat_dim = F2 // 2
    tb = _pick_tile(B)
    wh = w_head.astype(jnp.bfloat16)
    wp = w_proj.astype(jnp.bfloat16)
    out_block = pl.BlockSpec((tb, feat_dim), lambda i: (i, 0))
    return pl.pallas_call(
        functools.partial(_body, feat_dim=feat_dim, tb=tb),
        out_shape=tuple(jax.ShapeDtypeStruct((B, feat_dim), x.dtype)
                        for _ in range(4)),
        grid=(B // tb,),
        in_specs=[
            pl.BlockSpec((tb, D), lambda i: (i, 0)),
            pl.BlockSpec((tb, D), lambda i: (i, 0)),
            pl.BlockSpec((D, D), lambda i: (0, 0)),
            pl.BlockSpec((1, D), lambda i: (0, 0)),
            pl.BlockSpec((D, F2), lambda i: (0, 0)),
            pl.BlockSpec((1, F2), lambda i: (0, 0)),
        ],
        out_specs=(out_block, out_block, out_block, out_block),
        compiler_params=pltpu.CompilerParams(
            dimension_semantics=("parallel",),
            vmem_limit_bytes=64 * 1024 * 1024,
        ),
    )(x, x2, wh, b_head, wp, b_proj)


def kernel(x, x2, w_head, b_head, w_proj, b_proj):
    return _run(x, x2, w_head, b_head, w_proj, b_proj)
